# Initial kernel scaffold; baseline (speedup 1.0000x reference)
import jax
import jax.numpy as jnp
from jax.experimental import pallas as pl


def kernel(features, point_to_site):
    raise NotImplementedError("write your pallas kernel here")


# speedup vs baseline: 4.2598x; 4.2598x over previous
import functools

import jax
import jax.numpy as jnp
from jax import lax
from jax.experimental import pallas as pl
from jax.experimental.pallas import tpu as pltpu
from jax.experimental.pallas import tpu_sc as plsc


def _make_gather(n_sites, d_feat, n_points):
    info = plsc.get_sparse_core_info()
    nc, ns = info.num_cores, info.num_subcores
    nw = nc * ns

    chunk = 784
    per_w = -(-n_points // nw)
    per_w = -(-per_w // chunk) * chunk
    k = per_w // chunk

    mesh = plsc.VectorSubcoreMesh(core_axis_name="c", subcore_axis_name="s")

    @functools.partial(
        pl.kernel,
        mesh=mesh,
        out_type=jax.ShapeDtypeStruct((n_points, d_feat), jnp.float32),
        scratch_types=[
            pltpu.VMEM((chunk,), jnp.int32),
            pltpu.VMEM((chunk, d_feat), jnp.float32),
            pltpu.SemaphoreType.DMA,
        ],
    )
    def gather_kernel(table_hbm, idx_hbm, out_hbm, idx_v, rows_v, sem):
        wid = lax.axis_index("s") * nc + lax.axis_index("c")
        base = wid * per_w
        for j in range(k):
            off = jnp.minimum(base + j * chunk, n_points - chunk)
            pltpu.sync_copy(idx_hbm.at[pl.ds(off, chunk)], idx_v)
            pltpu.async_copy(table_hbm.at[idx_v], rows_v, sem).wait()
            pltpu.sync_copy(rows_v, out_hbm.at[pl.ds(off, chunk)])

    return gather_kernel


def kernel(features, point_to_site):
    n_sites, d_feat = features.shape
    (n_points,) = point_to_site.shape
    return _make_gather(n_sites, d_feat, n_points)(features, point_to_site)


# speedup vs baseline: 4.3573x; 1.0229x over previous
import functools

import jax
import jax.numpy as jnp
from jax import lax
from jax.experimental import pallas as pl
from jax.experimental.pallas import tpu as pltpu
from jax.experimental.pallas import tpu_sc as plsc


def _make_gather(n_sites, d_feat, n_points):
    info = plsc.get_sparse_core_info()
    nc, ns = info.num_cores, info.num_subcores
    nw = nc * ns

    chunk = 448
    per_w = -(-n_points // nw)
    per_w = -(-per_w // chunk) * chunk
    k = per_w // chunk

    mesh = plsc.VectorSubcoreMesh(core_axis_name="c", subcore_axis_name="s")

    @functools.partial(
        pl.kernel,
        mesh=mesh,
        out_type=jax.ShapeDtypeStruct((n_points, d_feat), jnp.float32),
        scratch_types=[
            pltpu.VMEM((chunk,), jnp.int32),
            pltpu.VMEM((chunk,), jnp.int32),
            pltpu.VMEM((chunk, d_feat), jnp.float32),
            pltpu.VMEM((chunk, d_feat), jnp.float32),
            pltpu.SemaphoreType.DMA,
            pltpu.SemaphoreType.DMA,
            pltpu.SemaphoreType.DMA,
            pltpu.SemaphoreType.DMA,
        ],
    )
    def gather_kernel(table_hbm, idx_hbm, out_hbm, idx_v0, idx_v1,
                      rows_v0, rows_v1, gsem0, gsem1, ssem0, ssem1):
        wid = lax.axis_index("s") * nc + lax.axis_index("c")
        base = wid * per_w
        idx_v = (idx_v0, idx_v1)
        rows_v = (rows_v0, rows_v1)
        gsem = (gsem0, gsem1)
        ssem = (ssem0, ssem1)

        def chunk_off(j):
            return jnp.minimum(base + j * chunk, n_points - chunk)

        pltpu.sync_copy(idx_hbm.at[pl.ds(chunk_off(0), chunk)], idx_v[0])
        gathers = [pltpu.async_copy(table_hbm.at[idx_v[0]],
                                    rows_v[0], gsem[0]), None]
        stores = [None, None]
        for j in range(k):
            p = j % 2
            q = (j + 1) % 2
            if j + 1 < k:
                if stores[q] is not None:
                    stores[q].wait()
                pltpu.sync_copy(idx_hbm.at[pl.ds(chunk_off(j + 1), chunk)],
                                idx_v[q])
                gathers[q] = pltpu.async_copy(table_hbm.at[idx_v[q]],
                                              rows_v[q], gsem[q])
            gathers[p].wait()
            stores[p] = pltpu.async_copy(
                rows_v[p], out_hbm.at[pl.ds(chunk_off(j), chunk)], ssem[p])
        for st in stores:
            if st is not None:
                st.wait()

    return gather_kernel


def kernel(features, point_to_site):
    n_sites, d_feat = features.shape
    (n_points,) = point_to_site.shape
    return _make_gather(n_sites, d_feat, n_points)(features, point_to_site)


# speedup vs baseline: 4.3598x; 1.0006x over previous
import functools

import jax
import jax.numpy as jnp
from jax import lax
from jax.experimental import pallas as pl
from jax.experimental.pallas import tpu as pltpu
from jax.experimental.pallas import tpu_sc as plsc


def _make_gather(n_sites, d_feat, n_points):
    info = plsc.get_sparse_core_info()
    nc, ns = info.num_cores, info.num_subcores
    nw = nc * ns

    chunk = 448
    per_w = -(-n_points // nw)
    per_w = -(-per_w // chunk) * chunk
    k = per_w // chunk

    mesh = plsc.VectorSubcoreMesh(core_axis_name="c", subcore_axis_name="s")

    @functools.partial(
        pl.kernel,
        mesh=mesh,
        out_type=jax.ShapeDtypeStruct((n_points, d_feat), jnp.float32),
        scratch_types=[
            pltpu.VMEM((per_w,), jnp.int32),
            pltpu.VMEM((chunk, d_feat), jnp.float32),
            pltpu.VMEM((chunk, d_feat), jnp.float32),
            pltpu.SemaphoreType.DMA,
            pltpu.SemaphoreType.DMA,
            pltpu.SemaphoreType.DMA,
            pltpu.SemaphoreType.DMA,
        ],
    )
    def gather_kernel(table_hbm, idx_hbm, out_hbm, idx_v,
                      rows_v0, rows_v1, gsem0, gsem1, ssem0, ssem1):
        wid = lax.axis_index("s") * nc + lax.axis_index("c")
        base = jnp.minimum(wid * per_w, n_points - per_w)
        rows_v = (rows_v0, rows_v1)
        gsem = (gsem0, gsem1)
        ssem = (ssem0, ssem1)

        pltpu.sync_copy(idx_hbm.at[pl.ds(base, per_w)], idx_v)

        gathers = [pltpu.async_copy(
            table_hbm.at[idx_v.at[pl.ds(0, chunk)]], rows_v[0], gsem[0]),
            None]
        stores = [None, None]
        for j in range(k):
            p = j % 2
            q = (j + 1) % 2
            if j + 1 < k:
                if stores[q] is not None:
                    stores[q].wait()
                gathers[q] = pltpu.async_copy(
                    table_hbm.at[idx_v.at[pl.ds((j + 1) * chunk, chunk)]],
                    rows_v[q], gsem[q])
            gathers[p].wait()
            stores[p] = pltpu.async_copy(
                rows_v[p], out_hbm.at[pl.ds(base + j * chunk, chunk)],
                ssem[p])
        for st in stores:
            if st is not None:
                st.wait()

    return gather_kernel


def kernel(features, point_to_site):
    n_sites, d_feat = features.shape
    (n_points,) = point_to_site.shape
    return _make_gather(n_sites, d_feat, n_points)(features, point_to_site)


# speedup vs baseline: 4.3912x; 1.0072x over previous
import functools

import jax
import jax.numpy as jnp
from jax import lax
from jax.experimental import pallas as pl
from jax.experimental.pallas import tpu as pltpu
from jax.experimental.pallas import tpu_sc as plsc

_CHUNK = 224
_NBUF = 4


def _make_gather(n_sites, d_feat, n_points):
    info = plsc.get_sparse_core_info()
    nc, ns = info.num_cores, info.num_subcores
    nw = nc * ns

    chunk = _CHUNK
    nbuf = _NBUF
    per_w = -(-n_points // nw)
    per_w = -(-per_w // chunk) * chunk
    k = per_w // chunk

    mesh = plsc.VectorSubcoreMesh(core_axis_name="c", subcore_axis_name="s")

    scratch = [pltpu.VMEM((per_w,), jnp.int32)]
    scratch += [pltpu.VMEM((chunk, d_feat), jnp.float32)] * nbuf
    scratch += [pltpu.SemaphoreType.DMA] * (2 * nbuf)

    @functools.partial(
        pl.kernel,
        mesh=mesh,
        out_type=jax.ShapeDtypeStruct((n_points, d_feat), jnp.float32),
        scratch_types=scratch,
    )
    def gather_kernel(table_hbm, idx_hbm, out_hbm, idx_v, *bufs_sems):
        rows_v = bufs_sems[:nbuf]
        gsem = bufs_sems[nbuf:2 * nbuf]
        ssem = bufs_sems[2 * nbuf:]
        wid = lax.axis_index("s") * nc + lax.axis_index("c")
        base = jnp.minimum(wid * per_w, n_points - per_w)

        pltpu.sync_copy(idx_hbm.at[pl.ds(base, per_w)], idx_v)

        look = nbuf // 2
        gathers = [None] * nbuf
        stores = [None] * nbuf
        for j in range(min(look, k)):
            gathers[j] = pltpu.async_copy(
                table_hbm.at[idx_v.at[pl.ds(j * chunk, chunk)]],
                rows_v[j], gsem[j])
        for j in range(k):
            p = j % nbuf
            jn = j + look
            if jn < k:
                q = jn % nbuf
                if stores[q] is not None:
                    stores[q].wait()
                gathers[q] = pltpu.async_copy(
                    table_hbm.at[idx_v.at[pl.ds(jn * chunk, chunk)]],
                    rows_v[q], gsem[q])
            gathers[p].wait()
            stores[p] = pltpu.async_copy(
                rows_v[p], out_hbm.at[pl.ds(base + j * chunk, chunk)],
                ssem[p])
        for j in range(max(0, k - nbuf), k):
            if stores[j % nbuf] is not None:
                stores[j % nbuf].wait()

    return gather_kernel


def kernel(features, point_to_site):
    n_sites, d_feat = features.shape
    (n_points,) = point_to_site.shape
    return _make_gather(n_sites, d_feat, n_points)(features, point_to_site)
